# baseline (speedup 1.0000x reference)
import functools

import jax
import jax.numpy as jnp
from jax import lax
from jax.experimental import pallas as pl
from jax.experimental.pallas import tpu as pltpu
from jax.experimental.pallas import tpu_sc as plsc

EPS = 1e-5
NT = 16
NSC = 2
CB = 128
NB = 32
NR = 8
LAG = 4
ZR = 32


def _mlp_in_body(x_ref, w1_ref, b1_ref, g1_ref, t1_ref,
                 w2_ref, b2_ref, g2_ref, t2_ref, out_ref):
    s = (1.0 + EPS) ** -0.5
    xb = x_ref[...]
    h1 = jnp.dot(xb, w1_ref[...], preferred_element_type=jnp.float32)
    h1 = jnp.maximum((h1 + b1_ref[...]) * (s * g1_ref[...]) + t1_ref[...], 0.0)
    h2 = jnp.dot(h1, w2_ref[...], preferred_element_type=jnp.float32)
    h2 = jnp.maximum((h2 + b2_ref[...]) * (s * g2_ref[...]) + t2_ref[...], 0.0)
    h = h1 + h2
    f = out_ref.shape[2]
    out_ref[0] = h[:, :f]
    out_ref[1] = h[:, f:]


def _make_mlp_out_body(k_steps):
    def _mlp_out_body(st_ref, aw_ref, w3_ref, b3_ref, g3_ref, t3_ref,
                      wo_ref, bo_ref, out_ref):
        s = (1.0 + EPS) ** -0.5
        s0 = aw_ref[0, 0] * st_ref[0, 0]
        s1 = aw_ref[0, 0] * st_ref[0, 1]
        for k in range(1, k_steps + 1):
            s0 = s0 + aw_ref[0, k] * st_ref[k, 0]
            s1 = s1 + aw_ref[0, k] * st_ref[k, 1]
        h = jnp.concatenate([s0, s1], axis=1)
        h3 = jnp.dot(h, w3_ref[...], preferred_element_type=jnp.float32)
        h3 = jnp.maximum((h3 + b3_ref[...]) * (s * g3_ref[...]) + t3_ref[...], 0.0)
        out = jnp.dot(h3, wo_ref[...], preferred_element_type=jnp.float32)
        out_ref[...] = out + bo_ref[...]
    return _mlp_out_body


def _make_prop_kernel(npad, f, nch, k_steps, rt):
    nblk = nch // NB
    mesh = plsc.VectorSubcoreMesh(core_axis_name="c", subcore_axis_name="s")

    @functools.partial(
        pl.kernel,
        out_type=jax.ShapeDtypeStruct(((k_steps + 1) * NSC * npad, f),
                                      jnp.float32),
        mesh=mesh,
        compiler_params=pltpu.CompilerParams(use_tc_tiling_on_sc=False),
        scratch_types=[
            pltpu.VMEM((NB, CB), jnp.int32),
            pltpu.VMEM((NB, CB), jnp.int32),
            [pltpu.VMEM((CB, f), jnp.float32) for _ in range(NR)],
            pltpu.VMEM((ZR, f), jnp.float32),
            pltpu.VMEM_SHARED((npad, f), jnp.float32),
            [pltpu.SemaphoreType.DMA for _ in range(NR)],
        ],
    )
    def prop(h_hbm, src_hbm, dst_hbm, st_hbm,
             idx_src, idx_dst, rows, zeros, tab, sems):
        cidx = lax.axis_index("c")
        sidx = lax.axis_index("s")
        row0 = sidx * rt
        rsl = pl.ds(row0, rt)

        zvec = jnp.zeros((16,), jnp.float32)

        def zbody(r, c):
            for c4 in range(f // 16):
                zeros[r, pl.ds(c4 * 16, 16)] = zvec
            return c
        lax.fori_loop(0, ZR, zbody, 0)

        def zero_tab():
            for z in range(rt // ZR):
                pltpu.sync_copy(zeros, tab.at[pl.ds(row0 + z * ZR, ZR)])

        pltpu.sync_copy(h_hbm.at[cidx, rsl],
                        st_hbm.at[pl.ds(cidx * npad + row0, rt)])
        zero_tab()
        plsc.subcore_barrier()

        def kbody(kk, c):
            def bbody(b, c2):
                pltpu.sync_copy(src_hbm.at[kk, cidx, sidx, pl.ds(b * NB, NB)],
                                idx_src)
                pltpu.sync_copy(dst_hbm.at[sidx, pl.ds(b * NB, NB)], idx_dst)
                gd = [None] * NR
                sd = [None] * NR
                for j in range(NB):
                    bi = j % NR
                    if j >= NR:
                        sd[bi].wait()
                    gd[bi] = pltpu.async_copy(st_hbm.at[idx_src.at[j]],
                                              rows[bi], sems[bi])
                    if j >= LAG:
                        bj = (j - LAG) % NR
                        gd[bj].wait()
                        sd[bj] = pltpu.async_copy(rows[bj],
                                                  tab.at[idx_dst.at[j - LAG]],
                                                  sems[bj], add=True)
                for j in range(NB - LAG, NB):
                    bj = j % NR
                    gd[bj].wait()
                    sd[bj] = pltpu.async_copy(rows[bj], tab.at[idx_dst.at[j]],
                                              sems[bj], add=True)
                for j in range(NB - NR, NB):
                    sd[j % NR].wait()
                return c2
            lax.fori_loop(0, nblk, bbody, 0)
            plsc.subcore_barrier()

            off = ((kk + 1) * NSC + cidx) * npad + row0
            pltpu.sync_copy(tab.at[rsl], st_hbm.at[pl.ds(off, rt)])
            zero_tab()
            plsc.subcore_barrier()
            return c
        lax.fori_loop(0, k_steps, kbody, 0)

    return prop


def kernel(x, edge_index, W1, b1, g1, bt1, W2, b2, g2, bt2, att,
           W3, b3, g3, bt3, Wout, bout):
    n, d = x.shape
    h_dim = W1.shape[1]
    f = h_dim // 2
    f2 = W3.shape[1]
    o_dim = Wout.shape[1]
    e = edge_index.shape[1]
    k_steps = att.shape[0] - 1

    rt = -(-n // (NT * ZR)) * ZR
    npad = NT * rt
    nch = -(-e // (NT * CB * NB)) * NB
    e_pad = NT * nch * CB
    dummy = npad - 1

    xp = jnp.pad(x, ((0, npad - n), (0, 0)))
    src = jnp.concatenate(
        [edge_index[0], jnp.full((e_pad - e,), dummy, jnp.int32)]
    ).reshape(NT, nch, CB)
    dst = jnp.concatenate(
        [edge_index[1], jnp.full((e_pad - e,), dummy, jnp.int32)]
    ).reshape(NT, nch, CB)
    offs = ((jnp.arange(k_steps, dtype=jnp.int32) * NSC)[:, None]
            + jnp.arange(NSC, dtype=jnp.int32)[None, :]) * npad
    src_all = offs[:, :, None, None, None] + src[None, None]
    aw = jax.nn.softmax(att, axis=0)
    aw_pad = jnp.zeros((1, 128), jnp.float32).at[0, :k_steps + 1].set(aw)

    bn = 512
    grid = npad // bn
    row_spec = pl.BlockSpec((bn, d), lambda i: (i, 0))
    full = lambda shp: pl.BlockSpec(shp, lambda i: tuple(0 for _ in shp))
    h_split = pl.pallas_call(
        _mlp_in_body,
        grid=(grid,),
        in_specs=[
            row_spec,
            full((d, h_dim)), full((1, h_dim)), full((1, h_dim)), full((1, h_dim)),
            full((h_dim, h_dim)), full((1, h_dim)), full((1, h_dim)), full((1, h_dim)),
        ],
        out_specs=pl.BlockSpec((NSC, bn, f), lambda i: (0, i, 0)),
        out_shape=jax.ShapeDtypeStruct((NSC, npad, f), jnp.float32),
    )(xp, W1, b1.reshape(1, -1), g1.reshape(1, -1), bt1.reshape(1, -1),
      W2, b2.reshape(1, -1), g2.reshape(1, -1), bt2.reshape(1, -1))

    st_flat = _make_prop_kernel(npad, f, nch, k_steps, rt)(h_split, src_all, dst)
    states = st_flat.reshape(k_steps + 1, NSC, npad, f)

    out_full = pl.pallas_call(
        _make_mlp_out_body(k_steps),
        grid=(grid,),
        in_specs=[
            pl.BlockSpec((k_steps + 1, NSC, bn, f), lambda i: (0, 0, i, 0)),
            full((1, 128)),
            full((h_dim, f2)), full((1, f2)), full((1, f2)), full((1, f2)),
            full((f2, o_dim)), full((1, o_dim)),
        ],
        out_specs=pl.BlockSpec((bn, o_dim), lambda i: (i, 0)),
        out_shape=jax.ShapeDtypeStruct((npad, o_dim), jnp.float32),
    )(states, aw_pad,
      W3, b3.reshape(1, -1), g3.reshape(1, -1), bt3.reshape(1, -1),
      Wout, bout.reshape(1, -1))

    return out_full[:n]


# speedup vs baseline: 2.0649x; 2.0649x over previous
import functools

import jax
import jax.numpy as jnp
from jax import lax
from jax.experimental import pallas as pl
from jax.experimental.pallas import tpu as pltpu
from jax.experimental.pallas import tpu_sc as plsc

EPS = 1e-5
NT = 16
NSC = 2
CB = 128
NB = 16
NR = 4
LAG = 2
ZR = 32


def _mlp_in_body(x_ref, w1_ref, b1_ref, g1_ref, t1_ref,
                 w2_ref, b2_ref, g2_ref, t2_ref, out_ref):
    s = (1.0 + EPS) ** -0.5
    xb = x_ref[...]
    h1 = jnp.dot(xb, w1_ref[...], preferred_element_type=jnp.float32)
    h1 = jnp.maximum((h1 + b1_ref[...]) * (s * g1_ref[...]) + t1_ref[...], 0.0)
    h2 = jnp.dot(h1, w2_ref[...], preferred_element_type=jnp.float32)
    h2 = jnp.maximum((h2 + b2_ref[...]) * (s * g2_ref[...]) + t2_ref[...], 0.0)
    h = h1 + h2
    f = out_ref.shape[2]
    out_ref[0] = h[:, :f]
    out_ref[1] = h[:, f:]


def _make_mlp_out_body(k_steps):
    def _mlp_out_body(h0_ref, xs_ref, aw_ref, w3_ref, b3_ref, g3_ref, t3_ref,
                      wo_ref, bo_ref, out_ref):
        s = (1.0 + EPS) ** -0.5
        s0 = aw_ref[0, 0] * h0_ref[0]
        s1 = aw_ref[0, 0] * h0_ref[1]
        for k in range(k_steps):
            s0 = s0 + aw_ref[0, k + 1] * xs_ref[k, 0]
            s1 = s1 + aw_ref[0, k + 1] * xs_ref[k, 1]
        h = jnp.concatenate([s0, s1], axis=1)
        h3 = jnp.dot(h, w3_ref[...], preferred_element_type=jnp.float32)
        h3 = jnp.maximum((h3 + b3_ref[...]) * (s * g3_ref[...]) + t3_ref[...], 0.0)
        out = jnp.dot(h3, wo_ref[...], preferred_element_type=jnp.float32)
        out_ref[...] = out + bo_ref[...]
    return _mlp_out_body


def _make_prop_kernel(npad, f, nch, k_steps, rt):
    nblk = nch // NB
    mesh = plsc.VectorSubcoreMesh(core_axis_name="c", subcore_axis_name="s")

    @functools.partial(
        pl.kernel,
        out_type=jax.ShapeDtypeStruct((k_steps, NSC, npad, f), jnp.float32),
        mesh=mesh,
        compiler_params=pltpu.CompilerParams(use_tc_tiling_on_sc=False),
        scratch_types=[
            pltpu.VMEM((2, NB, 2, CB), jnp.int32),
            [pltpu.VMEM((CB, f), jnp.float32) for _ in range(NR)],
            pltpu.VMEM((ZR, f), jnp.float32),
            pltpu.VMEM_SHARED((npad, f), jnp.float32),
            pltpu.VMEM_SHARED((npad, f), jnp.float32),
            [pltpu.SemaphoreType.DMA for _ in range(NR)],
            pltpu.SemaphoreType.DMA,
        ],
    )
    def prop(h_hbm, ed_hbm, out_hbm,
             idx_v, rows, zeros, tab_a, tab_b, sems, sem_i):
        cidx = lax.axis_index("c")
        sidx = lax.axis_index("s")
        row0 = sidx * rt

        zvec = jnp.zeros((16,), jnp.float32)

        def zbody(r, c):
            for c4 in range(f // 16):
                zeros[r, pl.ds(c4 * 16, 16)] = zvec
            return c
        lax.fori_loop(0, ZR, zbody, 0)

        pltpu.sync_copy(h_hbm.at[cidx, pl.ds(row0, rt)], tab_a.at[pl.ds(row0, rt)])
        for z in range(rt // ZR):
            pltpu.sync_copy(zeros, tab_b.at[pl.ds(row0 + z * ZR, ZR)])
        plsc.subcore_barrier()

        def edge_pass(cur, nxt):
            pltpu.async_copy(ed_hbm.at[sidx, pl.ds(0, NB)], idx_v.at[0], sem_i)

            def bbody(b, c):
                p = lax.rem(b, 2)
                pltpu.make_async_copy(ed_hbm.at[sidx, pl.ds(b * NB, NB)],
                                      idx_v.at[p], sem_i).wait()
                nb1 = lax.rem(b + 1, nblk)
                pltpu.async_copy(ed_hbm.at[sidx, pl.ds(nb1 * NB, NB)],
                                 idx_v.at[1 - p], sem_i)
                gd = [None] * NR
                sd = [None] * NR
                for j in range(NB):
                    bi = j % NR
                    if j >= NR:
                        sd[bi].wait()
                    gd[bi] = pltpu.async_copy(cur.at[idx_v.at[p, j, 0]],
                                              rows[bi], sems[bi])
                    if j >= LAG:
                        bj = (j - LAG) % NR
                        gd[bj].wait()
                        sd[bj] = pltpu.async_copy(rows[bj],
                                                  nxt.at[idx_v.at[p, j - LAG, 1]],
                                                  sems[bj], add=True)
                for j in range(NB - LAG, NB):
                    bj = j % NR
                    gd[bj].wait()
                    sd[bj] = pltpu.async_copy(rows[bj],
                                              nxt.at[idx_v.at[p, j, 1]],
                                              sems[bj], add=True)
                for j in range(NB - NR, NB):
                    sd[j % NR].wait()
                return c
            lax.fori_loop(0, nblk, bbody, 0)
            pltpu.make_async_copy(ed_hbm.at[sidx, pl.ds(0, NB)],
                                  idx_v.at[lax.rem(nblk, 2)], sem_i).wait()
            plsc.subcore_barrier()

        def zero_pass(tab):
            for z in range(rt // ZR):
                pltpu.sync_copy(zeros, tab.at[pl.ds(row0 + z * ZR, ZR)])
            plsc.subcore_barrier()

        rsl = pl.ds(row0, rt)

        def kbody(i2, c):
            edge_pass(tab_a, tab_b)
            pltpu.sync_copy(tab_b.at[rsl], out_hbm.at[2 * i2, cidx, rsl])
            zero_pass(tab_a)
            edge_pass(tab_b, tab_a)
            pltpu.sync_copy(tab_a.at[rsl], out_hbm.at[2 * i2 + 1, cidx, rsl])
            zero_pass(tab_b)
            return c
        lax.fori_loop(0, k_steps // 2, kbody, 0)
        if k_steps % 2:
            edge_pass(tab_a, tab_b)
            pltpu.sync_copy(tab_b.at[rsl], out_hbm.at[k_steps - 1, cidx, rsl])

    return prop


def kernel(x, edge_index, W1, b1, g1, bt1, W2, b2, g2, bt2, att,
           W3, b3, g3, bt3, Wout, bout):
    n, d = x.shape
    h_dim = W1.shape[1]
    f = h_dim // 2
    f2 = W3.shape[1]
    o_dim = Wout.shape[1]
    e = edge_index.shape[1]
    k_steps = att.shape[0] - 1

    rt = -(-n // (NT * ZR)) * ZR
    npad = NT * rt
    nch = -(-e // (NT * CB * NB)) * NB
    e_pad = NT * nch * CB
    dummy = npad - 1

    xp = jnp.pad(x, ((0, npad - n), (0, 0)))
    src = jnp.concatenate(
        [edge_index[0], jnp.full((e_pad - e,), dummy, jnp.int32)]
    ).reshape(NT, nch, CB)
    dst = jnp.concatenate(
        [edge_index[1], jnp.full((e_pad - e,), dummy, jnp.int32)]
    ).reshape(NT, nch, CB)
    ed = jnp.stack([src, dst], axis=2)
    aw = jax.nn.softmax(att, axis=0)
    aw_pad = jnp.zeros((1, 128), jnp.float32).at[0, :k_steps + 1].set(aw)

    bn = 512
    grid = npad // bn
    row_spec = pl.BlockSpec((bn, d), lambda i: (i, 0))
    full = lambda shp: pl.BlockSpec(shp, lambda i: tuple(0 for _ in shp))
    h_split = pl.pallas_call(
        _mlp_in_body,
        grid=(grid,),
        in_specs=[
            row_spec,
            full((d, h_dim)), full((1, h_dim)), full((1, h_dim)), full((1, h_dim)),
            full((h_dim, h_dim)), full((1, h_dim)), full((1, h_dim)), full((1, h_dim)),
        ],
        out_specs=pl.BlockSpec((NSC, bn, f), lambda i: (0, i, 0)),
        out_shape=jax.ShapeDtypeStruct((NSC, npad, f), jnp.float32),
    )(xp, W1, b1.reshape(1, -1), g1.reshape(1, -1), bt1.reshape(1, -1),
      W2, b2.reshape(1, -1), g2.reshape(1, -1), bt2.reshape(1, -1))

    xs = _make_prop_kernel(npad, f, nch, k_steps, rt)(h_split, ed)

    out_full = pl.pallas_call(
        _make_mlp_out_body(k_steps),
        grid=(grid,),
        in_specs=[
            pl.BlockSpec((NSC, bn, f), lambda i: (0, i, 0)),
            pl.BlockSpec((k_steps, NSC, bn, f), lambda i: (0, 0, i, 0)),
            full((1, 128)),
            full((h_dim, f2)), full((1, f2)), full((1, f2)), full((1, f2)),
            full((f2, o_dim)), full((1, o_dim)),
        ],
        out_specs=pl.BlockSpec((bn, o_dim), lambda i: (i, 0)),
        out_shape=jax.ShapeDtypeStruct((npad, o_dim), jnp.float32),
    )(h_split, xs, aw_pad,
      W3, b3.reshape(1, -1), g3.reshape(1, -1), bt3.reshape(1, -1),
      Wout, bout.reshape(1, -1))

    return out_full[:n]


# speedup vs baseline: 2.1567x; 1.0445x over previous
import functools

import jax
import jax.numpy as jnp
from jax import lax
from jax.experimental import pallas as pl
from jax.experimental.pallas import tpu as pltpu
from jax.experimental.pallas import tpu_sc as plsc

EPS = 1e-5
NT = 16
NSC = 2
CB = 128
NB = 32
NR = 4
LAG = 2
ZR = 32


def _mlp_in_body(x_ref, w1_ref, b1_ref, g1_ref, t1_ref,
                 w2_ref, b2_ref, g2_ref, t2_ref, out_ref):
    s = (1.0 + EPS) ** -0.5
    xb = x_ref[...]
    h1 = jnp.dot(xb, w1_ref[...], preferred_element_type=jnp.float32)
    h1 = jnp.maximum((h1 + b1_ref[...]) * (s * g1_ref[...]) + t1_ref[...], 0.0)
    h2 = jnp.dot(h1, w2_ref[...], preferred_element_type=jnp.float32)
    h2 = jnp.maximum((h2 + b2_ref[...]) * (s * g2_ref[...]) + t2_ref[...], 0.0)
    h = h1 + h2
    f = out_ref.shape[2]
    out_ref[0] = h[:, :f]
    out_ref[1] = h[:, f:]


def _make_mlp_out_body(k_steps):
    def _mlp_out_body(h0_ref, xs_ref, aw_ref, w3_ref, b3_ref, g3_ref, t3_ref,
                      wo_ref, bo_ref, out_ref):
        s = (1.0 + EPS) ** -0.5
        s0 = aw_ref[0, 0] * h0_ref[0]
        s1 = aw_ref[0, 0] * h0_ref[1]
        for k in range(k_steps):
            s0 = s0 + aw_ref[0, k + 1] * xs_ref[k, 0]
            s1 = s1 + aw_ref[0, k + 1] * xs_ref[k, 1]
        h = jnp.concatenate([s0, s1], axis=1)
        h3 = jnp.dot(h, w3_ref[...], preferred_element_type=jnp.float32)
        h3 = jnp.maximum((h3 + b3_ref[...]) * (s * g3_ref[...]) + t3_ref[...], 0.0)
        out = jnp.dot(h3, wo_ref[...], preferred_element_type=jnp.float32)
        out_ref[...] = out + bo_ref[...]
    return _mlp_out_body


def _make_prop_kernel(npad, f, nch, k_steps, rt):
    nblk = nch // NB
    mesh = plsc.VectorSubcoreMesh(core_axis_name="c", subcore_axis_name="s")

    @functools.partial(
        pl.kernel,
        out_type=jax.ShapeDtypeStruct((k_steps, NSC, npad, f), jnp.float32),
        mesh=mesh,
        compiler_params=pltpu.CompilerParams(use_tc_tiling_on_sc=False),
        scratch_types=[
            pltpu.VMEM((NB, 2, CB), jnp.int32),
            [pltpu.VMEM((CB, f), jnp.float32) for _ in range(NR)],
            pltpu.VMEM((ZR, f), jnp.float32),
            pltpu.VMEM_SHARED((npad, f), jnp.float32),
            pltpu.VMEM_SHARED((npad, f), jnp.float32),
            [pltpu.SemaphoreType.DMA for _ in range(NR)],
        ],
    )
    def prop(h_hbm, ed_hbm, out_hbm,
             idx_v, rows, zeros, tab_a, tab_b, sems):
        cidx = lax.axis_index("c")
        sidx = lax.axis_index("s")
        row0 = sidx * rt

        zvec = jnp.zeros((16,), jnp.float32)

        def zbody(r, c):
            for c4 in range(f // 16):
                zeros[r, pl.ds(c4 * 16, 16)] = zvec
            return c
        lax.fori_loop(0, ZR, zbody, 0)

        pltpu.sync_copy(h_hbm.at[cidx, pl.ds(row0, rt)], tab_a.at[pl.ds(row0, rt)])
        for z in range(rt // ZR):
            pltpu.sync_copy(zeros, tab_b.at[pl.ds(row0 + z * ZR, ZR)])
        plsc.subcore_barrier()

        def edge_pass(cur, nxt):
            def bbody(b, c):
                pltpu.sync_copy(ed_hbm.at[sidx, pl.ds(b * NB, NB)], idx_v)
                gd = [None] * NR
                sd = [None] * NR
                for j in range(NB):
                    bi = j % NR
                    if j >= NR:
                        sd[bi].wait()
                    gd[bi] = pltpu.async_copy(cur.at[idx_v.at[j, 0]],
                                              rows[bi], sems[bi])
                    if j >= LAG:
                        bj = (j - LAG) % NR
                        gd[bj].wait()
                        sd[bj] = pltpu.async_copy(rows[bj],
                                                  nxt.at[idx_v.at[j - LAG, 1]],
                                                  sems[bj], add=True)
                for j in range(NB - LAG, NB):
                    bj = j % NR
                    gd[bj].wait()
                    sd[bj] = pltpu.async_copy(rows[bj],
                                              nxt.at[idx_v.at[j, 1]],
                                              sems[bj], add=True)
                for j in range(NB - NR, NB):
                    sd[j % NR].wait()
                return c
            lax.fori_loop(0, nblk, bbody, 0)
            plsc.subcore_barrier()

        def zero_pass(tab):
            for z in range(rt // ZR):
                pltpu.sync_copy(zeros, tab.at[pl.ds(row0 + z * ZR, ZR)])
            plsc.subcore_barrier()

        rsl = pl.ds(row0, rt)

        def kbody(i2, c):
            edge_pass(tab_a, tab_b)
            pltpu.sync_copy(tab_b.at[rsl], out_hbm.at[2 * i2, cidx, rsl])
            zero_pass(tab_a)
            edge_pass(tab_b, tab_a)
            pltpu.sync_copy(tab_a.at[rsl], out_hbm.at[2 * i2 + 1, cidx, rsl])
            zero_pass(tab_b)
            return c
        lax.fori_loop(0, k_steps // 2, kbody, 0)
        if k_steps % 2:
            edge_pass(tab_a, tab_b)
            pltpu.sync_copy(tab_b.at[rsl], out_hbm.at[k_steps - 1, cidx, rsl])

    return prop


def kernel(x, edge_index, W1, b1, g1, bt1, W2, b2, g2, bt2, att,
           W3, b3, g3, bt3, Wout, bout):
    n, d = x.shape
    h_dim = W1.shape[1]
    f = h_dim // 2
    f2 = W3.shape[1]
    o_dim = Wout.shape[1]
    e = edge_index.shape[1]
    k_steps = att.shape[0] - 1

    rt = -(-n // (NT * ZR)) * ZR
    npad = NT * rt
    nch = -(-e // (NT * CB * NB)) * NB
    e_pad = NT * nch * CB
    dummy = npad - 1

    xp = jnp.pad(x, ((0, npad - n), (0, 0)))
    src = jnp.concatenate(
        [edge_index[0], jnp.full((e_pad - e,), dummy, jnp.int32)]
    ).reshape(NT, nch, CB)
    dst = jnp.concatenate(
        [edge_index[1], jnp.full((e_pad - e,), dummy, jnp.int32)]
    ).reshape(NT, nch, CB)
    ed = jnp.stack([src, dst], axis=2)
    aw = jax.nn.softmax(att, axis=0)
    aw_pad = jnp.zeros((1, 128), jnp.float32).at[0, :k_steps + 1].set(aw)

    bn = 512
    grid = npad // bn
    row_spec = pl.BlockSpec((bn, d), lambda i: (i, 0))
    full = lambda shp: pl.BlockSpec(shp, lambda i: tuple(0 for _ in shp))
    h_split = pl.pallas_call(
        _mlp_in_body,
        grid=(grid,),
        in_specs=[
            row_spec,
            full((d, h_dim)), full((1, h_dim)), full((1, h_dim)), full((1, h_dim)),
            full((h_dim, h_dim)), full((1, h_dim)), full((1, h_dim)), full((1, h_dim)),
        ],
        out_specs=pl.BlockSpec((NSC, bn, f), lambda i: (0, i, 0)),
        out_shape=jax.ShapeDtypeStruct((NSC, npad, f), jnp.float32),
    )(xp, W1, b1.reshape(1, -1), g1.reshape(1, -1), bt1.reshape(1, -1),
      W2, b2.reshape(1, -1), g2.reshape(1, -1), bt2.reshape(1, -1))

    xs = _make_prop_kernel(npad, f, nch, k_steps, rt)(h_split, ed)

    out_full = pl.pallas_call(
        _make_mlp_out_body(k_steps),
        grid=(grid,),
        in_specs=[
            pl.BlockSpec((NSC, bn, f), lambda i: (0, i, 0)),
            pl.BlockSpec((k_steps, NSC, bn, f), lambda i: (0, 0, i, 0)),
            full((1, 128)),
            full((h_dim, f2)), full((1, f2)), full((1, f2)), full((1, f2)),
            full((f2, o_dim)), full((1, o_dim)),
        ],
        out_specs=pl.BlockSpec((bn, o_dim), lambda i: (i, 0)),
        out_shape=jax.ShapeDtypeStruct((npad, o_dim), jnp.float32),
    )(h_split, xs, aw_pad,
      W3, b3.reshape(1, -1), g3.reshape(1, -1), bt3.reshape(1, -1),
      Wout, bout.reshape(1, -1))

    return out_full[:n]


# speedup vs baseline: 2.1898x; 1.0154x over previous
import functools

import jax
import jax.numpy as jnp
from jax import lax
from jax.experimental import pallas as pl
from jax.experimental.pallas import tpu as pltpu
from jax.experimental.pallas import tpu_sc as plsc

EPS = 1e-5
NT = 16
NSC = 2
CB = 128
NB = 32
NR = 4
LAG = 2
ZR = 64


def _mlp_in_body(x_ref, w1_ref, b1_ref, g1_ref, t1_ref,
                 w2_ref, b2_ref, g2_ref, t2_ref, out_ref):
    s = (1.0 + EPS) ** -0.5
    xb = x_ref[...]
    h1 = jnp.dot(xb, w1_ref[...], preferred_element_type=jnp.float32)
    h1 = jnp.maximum((h1 + b1_ref[...]) * (s * g1_ref[...]) + t1_ref[...], 0.0)
    h2 = jnp.dot(h1, w2_ref[...], preferred_element_type=jnp.float32)
    h2 = jnp.maximum((h2 + b2_ref[...]) * (s * g2_ref[...]) + t2_ref[...], 0.0)
    h = h1 + h2
    f = out_ref.shape[2]
    out_ref[0] = h[:, :f]
    out_ref[1] = h[:, f:]


def _make_mlp_out_body(k_steps):
    def _mlp_out_body(h0_ref, xs_ref, aw_ref, w3_ref, b3_ref, g3_ref, t3_ref,
                      wo_ref, bo_ref, out_ref):
        s = (1.0 + EPS) ** -0.5
        s0 = aw_ref[0, 0] * h0_ref[0]
        s1 = aw_ref[0, 0] * h0_ref[1]
        for k in range(k_steps):
            s0 = s0 + aw_ref[0, k + 1] * xs_ref[k, 0]
            s1 = s1 + aw_ref[0, k + 1] * xs_ref[k, 1]
        h = jnp.concatenate([s0, s1], axis=1)
        h3 = jnp.dot(h, w3_ref[...], preferred_element_type=jnp.float32)
        h3 = jnp.maximum((h3 + b3_ref[...]) * (s * g3_ref[...]) + t3_ref[...], 0.0)
        out = jnp.dot(h3, wo_ref[...], preferred_element_type=jnp.float32)
        out_ref[...] = out + bo_ref[...]
    return _mlp_out_body


def _make_prop_kernel(npad, f, nch, k_steps, rt):
    nblk = nch // NB
    mesh = plsc.VectorSubcoreMesh(core_axis_name="c", subcore_axis_name="s")

    @functools.partial(
        pl.kernel,
        out_type=jax.ShapeDtypeStruct((k_steps, NSC, npad, f), jnp.float32),
        mesh=mesh,
        compiler_params=pltpu.CompilerParams(use_tc_tiling_on_sc=False),
        scratch_types=[
            pltpu.VMEM((NB, 2, CB), jnp.int32),
            [pltpu.VMEM((CB, f), jnp.float32) for _ in range(NR)],
            pltpu.VMEM((ZR, f), jnp.float32),
            pltpu.VMEM_SHARED((npad, f), jnp.float32),
            pltpu.VMEM_SHARED((npad, f), jnp.float32),
            [pltpu.SemaphoreType.DMA for _ in range(NR)],
        ],
    )
    def prop(h_hbm, ed_hbm, out_hbm,
             idx_v, rows, zeros, tab_a, tab_b, sems):
        cidx = lax.axis_index("c")
        sidx = lax.axis_index("s")
        row0 = sidx * rt

        zvec = jnp.zeros((16,), jnp.float32)

        def zbody(r, c):
            for c4 in range(f // 16):
                zeros[r, pl.ds(c4 * 16, 16)] = zvec
            return c
        lax.fori_loop(0, ZR, zbody, 0)

        pltpu.sync_copy(h_hbm.at[cidx, pl.ds(row0, rt)], tab_a.at[pl.ds(row0, rt)])
        for z in range(rt // ZR):
            pltpu.sync_copy(zeros, tab_b.at[pl.ds(row0 + z * ZR, ZR)])
        plsc.subcore_barrier()

        def edge_pass(cur, nxt):
            def bbody(b, c):
                pltpu.sync_copy(ed_hbm.at[sidx, pl.ds(b * NB, NB)], idx_v)
                gd = [None] * NR
                sd = [None] * NR
                for j in range(NB):
                    bi = j % NR
                    if j >= NR:
                        sd[bi].wait()
                    gd[bi] = pltpu.async_copy(cur.at[idx_v.at[j, 0]],
                                              rows[bi], sems[bi])
                    if j >= LAG:
                        bj = (j - LAG) % NR
                        gd[bj].wait()
                        sd[bj] = pltpu.async_copy(rows[bj],
                                                  nxt.at[idx_v.at[j - LAG, 1]],
                                                  sems[bj], add=True)
                for j in range(NB - LAG, NB):
                    bj = j % NR
                    gd[bj].wait()
                    sd[bj] = pltpu.async_copy(rows[bj],
                                              nxt.at[idx_v.at[j, 1]],
                                              sems[bj], add=True)
                for j in range(NB - NR, NB):
                    sd[j % NR].wait()
                return c
            lax.fori_loop(0, nblk, bbody, 0)
            plsc.subcore_barrier()

        rsl = pl.ds(row0, rt)

        def export_and_zero(done_tab, other_tab, kk):
            ex = pltpu.async_copy(done_tab.at[rsl], out_hbm.at[kk, cidx, rsl],
                                  sems[0])
            for z in range(rt // ZR):
                pltpu.sync_copy(zeros, other_tab.at[pl.ds(row0 + z * ZR, ZR)])
            ex.wait()
            plsc.subcore_barrier()

        def kbody(i2, c):
            edge_pass(tab_a, tab_b)
            export_and_zero(tab_b, tab_a, 2 * i2)
            edge_pass(tab_b, tab_a)
            export_and_zero(tab_a, tab_b, 2 * i2 + 1)
            return c
        lax.fori_loop(0, k_steps // 2, kbody, 0)
        if k_steps % 2:
            edge_pass(tab_a, tab_b)
            pltpu.sync_copy(tab_b.at[rsl], out_hbm.at[k_steps - 1, cidx, rsl])

    return prop


def kernel(x, edge_index, W1, b1, g1, bt1, W2, b2, g2, bt2, att,
           W3, b3, g3, bt3, Wout, bout):
    n, d = x.shape
    h_dim = W1.shape[1]
    f = h_dim // 2
    f2 = W3.shape[1]
    o_dim = Wout.shape[1]
    e = edge_index.shape[1]
    k_steps = att.shape[0] - 1

    rt = -(-n // (NT * ZR)) * ZR
    npad = NT * rt
    nch = -(-e // (NT * CB * NB)) * NB
    e_pad = NT * nch * CB
    dummy = npad - 1

    xp = jnp.pad(x, ((0, npad - n), (0, 0)))
    src = jnp.concatenate(
        [edge_index[0], jnp.full((e_pad - e,), dummy, jnp.int32)]
    ).reshape(NT, nch, CB)
    dst = jnp.concatenate(
        [edge_index[1], jnp.full((e_pad - e,), dummy, jnp.int32)]
    ).reshape(NT, nch, CB)
    ed = jnp.stack([src, dst], axis=2)
    aw = jax.nn.softmax(att, axis=0)
    aw_pad = jnp.zeros((1, 128), jnp.float32).at[0, :k_steps + 1].set(aw)

    bn = 512
    grid = npad // bn
    row_spec = pl.BlockSpec((bn, d), lambda i: (i, 0))
    full = lambda shp: pl.BlockSpec(shp, lambda i: tuple(0 for _ in shp))
    h_split = pl.pallas_call(
        _mlp_in_body,
        grid=(grid,),
        in_specs=[
            row_spec,
            full((d, h_dim)), full((1, h_dim)), full((1, h_dim)), full((1, h_dim)),
            full((h_dim, h_dim)), full((1, h_dim)), full((1, h_dim)), full((1, h_dim)),
        ],
        out_specs=pl.BlockSpec((NSC, bn, f), lambda i: (0, i, 0)),
        out_shape=jax.ShapeDtypeStruct((NSC, npad, f), jnp.float32),
    )(xp, W1, b1.reshape(1, -1), g1.reshape(1, -1), bt1.reshape(1, -1),
      W2, b2.reshape(1, -1), g2.reshape(1, -1), bt2.reshape(1, -1))

    xs = _make_prop_kernel(npad, f, nch, k_steps, rt)(h_split, ed)

    out_full = pl.pallas_call(
        _make_mlp_out_body(k_steps),
        grid=(grid,),
        in_specs=[
            pl.BlockSpec((NSC, bn, f), lambda i: (0, i, 0)),
            pl.BlockSpec((k_steps, NSC, bn, f), lambda i: (0, 0, i, 0)),
            full((1, 128)),
            full((h_dim, f2)), full((1, f2)), full((1, f2)), full((1, f2)),
            full((f2, o_dim)), full((1, o_dim)),
        ],
        out_specs=pl.BlockSpec((bn, o_dim), lambda i: (i, 0)),
        out_shape=jax.ShapeDtypeStruct((npad, o_dim), jnp.float32),
    )(h_split, xs, aw_pad,
      W3, b3.reshape(1, -1), g3.reshape(1, -1), bt3.reshape(1, -1),
      Wout, bout.reshape(1, -1))

    return out_full[:n]
